# baseline (speedup 1.0000x reference)
import jax
import jax.numpy as jnp
from jax.experimental import pallas as pl
from jax.experimental.pallas import tpu as pltpu

_ROWS, _COLS = 16384, 128
_BLK = 8192
_LOG2E = 1.4426950408889634
_LN2 = 0.6931471805599453


def _bce_mean_kernel(pred_ref, target_ref, out_ref, acc_ref):
    i = pl.program_id(0)
    x = pred_ref[...]
    z = target_ref[...]
    a = jnp.abs(x)
    xz = x * z
    r = jnp.log2(1.0 + jnp.exp2(a * (-_LOG2E)))
    ones = jnp.ones((8, _BLK), jnp.float32)

    def colsum(m):
        return jax.lax.dot_general(
            ones, m, (((1,), (0,)), ((), ())),
            preferred_element_type=jnp.float32,
        )

    part = (colsum(x + a) - 2.0 * colsum(xz)) + (2.0 * _LN2) * colsum(r)

    @pl.when(i == 0)
    def _():
        acc_ref[...] = jnp.zeros_like(acc_ref)

    acc_ref[...] += part

    @pl.when(i == pl.num_programs(0) - 1)
    def _():
        out_ref[0, 0] = jnp.sum(acc_ref[...]) * (1.0 / (16 * _ROWS * _COLS))


def kernel(pred, target):
    out = pl.pallas_call(
        _bce_mean_kernel,
        grid=(_ROWS // _BLK,),
        in_specs=[
            pl.BlockSpec((_BLK, _COLS), lambda i: (i, 0)),
            pl.BlockSpec((_BLK, _COLS), lambda i: (i, 0)),
        ],
        out_specs=pl.BlockSpec(memory_space=pltpu.SMEM),
        out_shape=jax.ShapeDtypeStruct((1, 1), jnp.float32),
        scratch_shapes=[pltpu.VMEM((8, _COLS), jnp.float32)],
        compiler_params=pltpu.CompilerParams(
            dimension_semantics=("arbitrary",),
        ),
    )(pred, target)
    return out[0, 0]


# speedup vs baseline: 1.0360x; 1.0360x over previous
import jax
import jax.numpy as jnp
from jax.experimental import pallas as pl
from jax.experimental.pallas import tpu as pltpu

_ROWS, _COLS = 16384, 128
_BLK = 8192


def _bce_mean_kernel(pred_ref, target_ref, out_ref, acc_ref):
    i = pl.program_id(0)
    x = pred_ref[...]
    z = target_ref[...]
    bce = jnp.maximum(x, 0.0) - x * z + jnp.log(1.0 + jnp.exp(-jnp.abs(x)))
    ones = jnp.ones((8, _BLK), jnp.float32)
    part = jax.lax.dot_general(
        ones, bce, (((1,), (0,)), ((), ())),
        preferred_element_type=jnp.float32,
    )

    @pl.when(i == 0)
    def _():
        acc_ref[...] = jnp.zeros_like(acc_ref)

    acc_ref[...] += part

    @pl.when(i == pl.num_programs(0) - 1)
    def _():
        out_ref[0, 0] = jnp.sum(acc_ref[...]) * (1.0 / (8 * _ROWS * _COLS))


def kernel(pred, target):
    out = pl.pallas_call(
        _bce_mean_kernel,
        grid=(_ROWS // _BLK,),
        in_specs=[
            pl.BlockSpec((_BLK, _COLS), lambda i: (i, 0)),
            pl.BlockSpec((_BLK, _COLS), lambda i: (i, 0)),
        ],
        out_specs=pl.BlockSpec(memory_space=pltpu.SMEM),
        out_shape=jax.ShapeDtypeStruct((1, 1), jnp.float32),
        scratch_shapes=[pltpu.VMEM((8, _COLS), jnp.float32)],
        compiler_params=pltpu.CompilerParams(
            dimension_semantics=("arbitrary",),
        ),
    )(pred, target)
    return out[0, 0]


# speedup vs baseline: 1.1199x; 1.0810x over previous
import jax
import jax.numpy as jnp
from jax.experimental import pallas as pl
from jax.experimental.pallas import tpu as pltpu

_ROWS, _COLS = 16384, 128
_CHUNK = 4096
_NCHUNK = _ROWS // _CHUNK


def _bce_mean_kernel(x_hbm, z_hbm, out_ref, xb, zb, sem):
    def x_copy(k):
        return pltpu.make_async_copy(
            x_hbm.at[pl.ds(k * _CHUNK, _CHUNK), :], xb.at[k], sem.at[k, 0])

    def z_copy(k):
        return pltpu.make_async_copy(
            z_hbm.at[pl.ds(k * _CHUNK, _CHUNK), :], zb.at[k], sem.at[k, 1])

    for k in range(_NCHUNK):
        x_copy(k).start()
        z_copy(k).start()

    ones = jnp.ones((8, _CHUNK), jnp.float32)
    acc = jnp.zeros((8, _COLS), jnp.float32)
    for k in range(_NCHUNK):
        x_copy(k).wait()
        z_copy(k).wait()
        x = xb[k]
        z = zb[k]
        bce = jnp.maximum(x, 0.0) - x * z + jnp.log(1.0 + jnp.exp(-jnp.abs(x)))
        acc = acc + jax.lax.dot_general(
            ones, bce, (((1,), (0,)), ((), ())),
            preferred_element_type=jnp.float32,
        )

    out_ref[0, 0] = jnp.sum(acc) * (1.0 / (8 * _ROWS * _COLS))


def kernel(pred, target):
    out = pl.pallas_call(
        _bce_mean_kernel,
        in_specs=[
            pl.BlockSpec(memory_space=pltpu.MemorySpace.HBM),
            pl.BlockSpec(memory_space=pltpu.MemorySpace.HBM),
        ],
        out_specs=pl.BlockSpec(memory_space=pltpu.SMEM),
        out_shape=jax.ShapeDtypeStruct((1, 1), jnp.float32),
        scratch_shapes=[
            pltpu.VMEM((_NCHUNK, _CHUNK, _COLS), jnp.float32),
            pltpu.VMEM((_NCHUNK, _CHUNK, _COLS), jnp.float32),
            pltpu.SemaphoreType.DMA((_NCHUNK, 2)),
        ],
    )(pred, target)
    return out[0, 0]


# speedup vs baseline: 1.1437x; 1.0213x over previous
import jax
import jax.numpy as jnp
from jax.experimental import pallas as pl
from jax.experimental.pallas import tpu as pltpu

_ROWS, _COLS = 16384, 128
_CHUNK = 2048
_NCHUNK = _ROWS // _CHUNK


def _bce_mean_kernel(x_hbm, z_hbm, out_ref, xb, zb, sem):
    def x_copy(k):
        return pltpu.make_async_copy(
            x_hbm.at[pl.ds(k * _CHUNK, _CHUNK), :], xb.at[k], sem.at[k, 0])

    def z_copy(k):
        return pltpu.make_async_copy(
            z_hbm.at[pl.ds(k * _CHUNK, _CHUNK), :], zb.at[k], sem.at[k, 1])

    for k in range(_NCHUNK):
        x_copy(k).start()
        z_copy(k).start()

    ones = jnp.ones((8, _CHUNK), jnp.float32)
    acc = jnp.zeros((8, _COLS), jnp.float32)
    for k in range(_NCHUNK):
        x_copy(k).wait()
        z_copy(k).wait()
        x = xb[k]
        z = zb[k]
        bce = jnp.maximum(x, 0.0) - x * z + jnp.log(1.0 + jnp.exp(-jnp.abs(x)))
        acc = acc + jax.lax.dot_general(
            ones, bce, (((1,), (0,)), ((), ())),
            preferred_element_type=jnp.float32,
        )

    out_ref[0, 0] = jnp.sum(acc) * (1.0 / (8 * _ROWS * _COLS))


def kernel(pred, target):
    out = pl.pallas_call(
        _bce_mean_kernel,
        in_specs=[
            pl.BlockSpec(memory_space=pltpu.MemorySpace.HBM),
            pl.BlockSpec(memory_space=pltpu.MemorySpace.HBM),
        ],
        out_specs=pl.BlockSpec(memory_space=pltpu.SMEM),
        out_shape=jax.ShapeDtypeStruct((1, 1), jnp.float32),
        scratch_shapes=[
            pltpu.VMEM((_NCHUNK, _CHUNK, _COLS), jnp.float32),
            pltpu.VMEM((_NCHUNK, _CHUNK, _COLS), jnp.float32),
            pltpu.SemaphoreType.DMA((_NCHUNK, 2)),
        ],
    )(pred, target)
    return out[0, 0]
